# baseline (speedup 1.0000x reference)
import functools

import jax
import jax.numpy as jnp
from jax import lax
from jax.experimental import pallas as pl
from jax.experimental.pallas import tpu as pltpu
from jax.experimental.pallas import tpu_sc as plsc

BATCH = 16384
DIM = 64
TILE = 8
NUM_CORES = 2
NUM_SUBCORES = 16
NUM_WORKERS = NUM_CORES * NUM_SUBCORES
B_PER_W = BATCH // NUM_WORKERS
GRP = 16
NGRP = B_PER_W // GRP

_mesh = plsc.VectorSubcoreMesh(core_axis_name="c", subcore_axis_name="s")


@functools.partial(
    pl.kernel,
    mesh=_mesh,
    out_type=jax.ShapeDtypeStruct((BATCH, DIM), jnp.float32),
    scratch_types=[
        pltpu.VMEM((B_PER_W,), jnp.int32),
        pltpu.VMEM((GRP * TILE, DIM), jnp.float32),
        pltpu.VMEM((GRP * TILE, DIM), jnp.float32),
        pltpu.VMEM((B_PER_W, DIM), jnp.float32),
        pltpu.SemaphoreType.DMA,
    ],
)
def _sc_gather(table_hbm, idx_hbm, out_hbm, idx_v, gbuf0, gbuf1, oslab, sem):
    wid = lax.axis_index("s") * NUM_CORES + lax.axis_index("c")
    base = wid * B_PER_W
    pltpu.sync_copy(idx_hbm.at[pl.ds(base, B_PER_W)], idx_v)

    def fire(g, gbuf):
        ids = idx_v[pl.ds(g * GRP, GRP)]
        tvec = (ids >> 3) << 3
        return [
            pltpu.async_copy(
                table_hbm.at[pl.ds(pl.multiple_of(tvec[lane], TILE), TILE)],
                gbuf.at[pl.ds(lane * TILE, TILE)],
                sem,
            )
            for lane in range(GRP)
        ]

    def select(g, gbuf, copies):
        ids = idx_v[pl.ds(g * GRP, GRP)]
        rvec = ids & 7
        for c in copies:
            c.wait()
        for lane in range(GRP):
            r = rvec[lane]
            for k in range(DIM // 16):
                oslab[g * GRP + lane, pl.ds(k * 16, 16)] = (
                    gbuf[lane * TILE + r, pl.ds(k * 16, 16)])

    def body(i, carry):
        c0 = fire(2 * i, gbuf0)
        c1 = fire(2 * i + 1, gbuf1)
        select(2 * i, gbuf0, c0)
        select(2 * i + 1, gbuf1, c1)
        return carry

    lax.fori_loop(0, NGRP // 2, body, 0)
    pltpu.sync_copy(oslab, out_hbm.at[pl.ds(base, B_PER_W)])


def kernel(embedding_weight, batch):
    idx = batch.astype(jnp.int32)
    return _sc_gather(embedding_weight, idx)


# speedup vs baseline: 2.1418x; 2.1418x over previous
import functools

import jax
import jax.numpy as jnp
from jax import lax
from jax.experimental import pallas as pl
from jax.experimental.pallas import tpu as pltpu
from jax.experimental.pallas import tpu_sc as plsc

BATCH = 16384
DIM = 64
ROWS = 1500000
LANE = 128
KTILE = 8
NUM_CORES = 2
NUM_SUBCORES = 16
NUM_WORKERS = NUM_CORES * NUM_SUBCORES
B_PER_W = BATCH // NUM_WORKERS
SUB = 4

_mesh = plsc.VectorSubcoreMesh(core_axis_name="c", subcore_axis_name="s")


@functools.partial(
    pl.kernel,
    mesh=_mesh,
    out_type=jax.ShapeDtypeStruct((DIM, BATCH), jnp.float32),
    scratch_types=[
        pltpu.VMEM((B_PER_W,), jnp.int32),
        pltpu.VMEM((SUB * DIM, LANE), jnp.float32),
        pltpu.VMEM((SUB * DIM, LANE), jnp.float32),
        pltpu.VMEM((DIM, B_PER_W), jnp.float32),
        pltpu.SemaphoreType.DMA,
        pltpu.SemaphoreType.DMA,
    ],
    compiler_params=pltpu.CompilerParams(needs_layout_passes=False),
)
def _sc_gather(table_hbm, idx_hbm, out_hbm, idx_v, bufa, bufb, slab,
               sema, semb):
    wid = lax.axis_index("s") * NUM_CORES + lax.axis_index("c")
    base = wid * B_PER_W
    pltpu.sync_copy(idx_hbm.at[pl.ds(base, B_PER_W)], idx_v)

    lane = lax.iota(jnp.int32, 16)
    bufs = (bufa, bufb)
    sems = (sema, semb)

    def fire(cvec, i):
        buf, sem = bufs[i % 2], sems[i % 2]
        copies = []
        for t in range(SUB):
            c = pl.multiple_of(cvec[SUB * i + t] * LANE, LANE)
            for kt in range(KTILE):
                copies.append(pltpu.async_copy(
                    table_hbm.at[pl.ds(kt * 8, 8), pl.ds(c, LANE)],
                    buf.at[pl.ds(t * DIM + kt * 8, 8), pl.ds(0, LANE)],
                    sem,
                ))
        return copies

    def select(g, jvec, i, copies):
        buf = bufs[i % 2]
        for cp in copies:
            cp.wait()
        for t in range(SUB):
            e = g * 16 + SUB * i + t
            col = lax.broadcast(jvec[SUB * i + t], (16,))
            for kq in range(DIM // 16):
                rows = t * DIM + kq * 16 + lane
                vals = plsc.load_gather(buf, [rows, col])
                plsc.store_scatter(slab, [kq * 16 + lane,
                                          lax.broadcast(e, (16,))], vals)

    def body(g, carry):
        ids = idx_v[pl.ds(g * 16, 16)]
        cvec = ids >> 7
        jvec = ids & (LANE - 1)
        c0 = fire(cvec, 0)
        c1 = fire(cvec, 1)
        select(g, jvec, 0, c0)
        c2 = fire(cvec, 2)
        select(g, jvec, 1, c1)
        c3 = fire(cvec, 3)
        select(g, jvec, 2, c2)
        select(g, jvec, 3, c3)
        return carry

    lax.fori_loop(0, B_PER_W // 16, body, 0)
    pltpu.sync_copy(slab, out_hbm.at[pl.ds(0, DIM), pl.ds(base, B_PER_W)])


def kernel(embedding_weight, batch):
    table_t = embedding_weight.T
    idx = batch.astype(jnp.int32)
    out_t = _sc_gather(table_t, idx)
    return out_t.T


# speedup vs baseline: 2.1444x; 1.0012x over previous
import functools

import jax
import jax.numpy as jnp
from jax import lax
from jax.experimental import pallas as pl
from jax.experimental.pallas import tpu as pltpu
from jax.experimental.pallas import tpu_sc as plsc

BATCH = 16384
DIM = 64
ROWS = 1500000
LANE = 128
KTILE = 8
NUM_CORES = 2
NUM_SUBCORES = 16
NUM_WORKERS = NUM_CORES * NUM_SUBCORES
B_PER_W = BATCH // NUM_WORKERS
SUB = 4

_mesh = plsc.VectorSubcoreMesh(core_axis_name="c", subcore_axis_name="s")


@functools.partial(
    pl.kernel,
    mesh=_mesh,
    out_type=jax.ShapeDtypeStruct((DIM, BATCH), jnp.float32),
    scratch_types=[
        pltpu.VMEM((B_PER_W,), jnp.int32),
        pltpu.VMEM((SUB * DIM, LANE), jnp.float32),
        pltpu.VMEM((SUB * DIM, LANE), jnp.float32),
        pltpu.VMEM((DIM, B_PER_W), jnp.float32),
        pltpu.SemaphoreType.DMA,
        pltpu.SemaphoreType.DMA,
    ],
    compiler_params=pltpu.CompilerParams(needs_layout_passes=False),
)
def _sc_gather(table_hbm, idx_hbm, out_hbm, idx_v, bufa, bufb, slab,
               sema, semb):
    wid = lax.axis_index("s") * NUM_CORES + lax.axis_index("c")
    base = wid * B_PER_W
    pltpu.sync_copy(idx_hbm.at[pl.ds(base, B_PER_W)], idx_v)

    lane = lax.iota(jnp.int32, 16)
    bufs = (bufa, bufb)
    sems = (sema, semb)

    def fire(cvec, i):
        buf, sem = bufs[i % 2], sems[i % 2]
        buf4 = buf.reshape(SUB, KTILE, 8, LANE)
        copies = []
        for t in range(SUB):
            c = pl.multiple_of(cvec[SUB * i + t] * LANE, LANE)
            copies.append(pltpu.async_copy(
                table_hbm.at[pl.ds(0, KTILE), pl.ds(0, 8), pl.ds(c, LANE)],
                buf4.at[t],
                sem,
            ))
        return copies

    def select(g, jvec, i, copies):
        buf = bufs[i % 2]
        for cp in copies:
            cp.wait()
        for t in range(SUB):
            e = g * 16 + SUB * i + t
            col = lax.broadcast(jvec[SUB * i + t], (16,))
            for kq in range(DIM // 16):
                rows = t * DIM + kq * 16 + lane
                vals = plsc.load_gather(buf, [rows, col])
                plsc.store_scatter(slab, [kq * 16 + lane,
                                          lax.broadcast(e, (16,))], vals)

    def body(g, carry):
        ids = idx_v[pl.ds(g * 16, 16)]
        cvec = ids >> 7
        jvec = ids & (LANE - 1)
        c0 = fire(cvec, 0)
        c1 = fire(cvec, 1)
        select(g, jvec, 0, c0)
        c2 = fire(cvec, 2)
        select(g, jvec, 1, c1)
        c3 = fire(cvec, 3)
        select(g, jvec, 2, c2)
        select(g, jvec, 3, c3)
        return carry

    lax.fori_loop(0, B_PER_W // 16, body, 0)
    pltpu.sync_copy(slab, out_hbm.at[pl.ds(0, DIM), pl.ds(base, B_PER_W)])


def kernel(embedding_weight, batch):
    table4 = embedding_weight.T.reshape(KTILE, 8, ROWS)
    idx = batch.astype(jnp.int32)
    out_t = _sc_gather(table4, idx)
    return out_t.T
